# baseline (speedup 1.0000x reference)
import jax
import jax.numpy as jnp
from jax.experimental import pallas as pl
from jax.experimental.pallas import tpu as pltpu

N = 4096
D = 256
R = 3
TILE = 128
NSTEPS = N // TILE
K = 4
OB = 4


def _fused_body(feature_hbm, adj_hbm, w_hbm, out_hbm,
                buf_ref, h_ref, feat_ref, w_ref, outb_ref,
                adj_sem, fw_sem, out_sem):
    def adj_copy(step, slot):
        return pltpu.make_async_copy(
            adj_hbm.at[:, pl.ds(step * TILE, TILE), :],
            buf_ref.at[slot],
            adj_sem.at[slot])

    def out_copy(step, oslot):
        return pltpu.make_async_copy(
            outb_ref.at[oslot],
            out_hbm.at[pl.ds(step * TILE, TILE), :],
            out_sem.at[oslot])

    for s in range(K):
        adj_copy(s, s).start()

    feat_copy = pltpu.make_async_copy(feature_hbm, feat_ref, fw_sem.at[0])
    w_copy = pltpu.make_async_copy(w_hbm, w_ref, fw_sem.at[1])
    feat_copy.start()
    w_copy.start()
    feat_copy.wait()
    w_copy.wait()
    h_ref[...] = jnp.tanh(
        jnp.dot(feat_ref[...], w_ref[...], preferred_element_type=jnp.float32))

    for step in range(NSTEPS):
        slot = step % K
        oslot = step % OB
        adj_copy(step, slot).wait()
        a = buf_ref[slot, 0] + buf_ref[slot, 1] + buf_ref[slot, 2]
        if step >= OB:
            out_copy(step - OB, oslot).wait()
        outb_ref[oslot] = jnp.dot(a, h_ref[...],
                                  preferred_element_type=jnp.float32)
        out_copy(step, oslot).start()
        if step + K < NSTEPS:
            adj_copy(step + K, slot).start()

    for step in range(NSTEPS - OB, NSTEPS):
        out_copy(step, step % OB).wait()


@jax.jit
def kernel(feature, same_type_adj, W, b):
    del b
    return pl.pallas_call(
        _fused_body,
        in_specs=[
            pl.BlockSpec(memory_space=pl.ANY),
            pl.BlockSpec(memory_space=pl.ANY),
            pl.BlockSpec(memory_space=pl.ANY),
        ],
        out_specs=pl.BlockSpec(memory_space=pl.ANY),
        out_shape=jax.ShapeDtypeStruct((N, D), jnp.float32),
        scratch_shapes=[
            pltpu.VMEM((K, R, TILE, N), jnp.float32),
            pltpu.VMEM((N, D), jnp.float32),
            pltpu.VMEM((N, D), jnp.float32),
            pltpu.VMEM((D, D), jnp.float32),
            pltpu.VMEM((OB, TILE, D), jnp.float32),
            pltpu.SemaphoreType.DMA((K,)),
            pltpu.SemaphoreType.DMA((2,)),
            pltpu.SemaphoreType.DMA((OB,)),
        ],
    )(feature, same_type_adj, W)
